# baseline (speedup 1.0000x reference)
import jax
import jax.numpy as jnp
from jax import lax
from jax.experimental import pallas as pl
from jax.experimental.pallas import tpu as pltpu
from jax.experimental.pallas import tpu_sc as plsc

N_CORES = 2
N_SUBCORES = 16
N_WORKERS = N_CORES * N_SUBCORES
BATCH = 16384
DIM = 32
ROWS_PER_W = BATCH // N_WORKERS
CHUNK = 128
N_CHUNKS = ROWS_PER_W // CHUNK
GROUPS = CHUNK // 16


def _gmf_body(user_h, item_h, ut_h, it_h, w_h, b_h, out_h,
              uidx, iidx, urow, irow, ucol, icol,
              ubuf, vbuf, wv, bv, outv, sem):
    wid = lax.axis_index("s") * N_CORES + lax.axis_index("c")
    base = wid * ROWS_PER_W

    pltpu.sync_copy(user_h.at[pl.ds(base, ROWS_PER_W)], uidx)
    pltpu.sync_copy(item_h.at[pl.ds(base, ROWS_PER_W)], iidx)
    pltpu.sync_copy(w_h, wv)
    pltpu.sync_copy(b_h, bv)

    for k in range(ROWS_PER_W // 16):
        sl = pl.ds(k * 16, 16)
        u = uidx[sl]
        i = iidx[sl]
        urow[sl] = lax.shift_right_logical(u, 2)
        irow[sl] = lax.shift_right_logical(i, 2)
        ucol[sl] = lax.shift_left(jnp.bitwise_and(u, 3), 5)
        icol[sl] = lax.shift_left(jnp.bitwise_and(i, 3), 5)

    wlo = wv[pl.ds(0, 16)]
    whi = wv[pl.ds(16, 16)]
    w_s = [wlo[d] for d in range(16)] + [whi[d] for d in range(16)]
    bvec = bv[...]
    iota16 = lax.iota(jnp.int32, 16)

    for j in range(N_CHUNKS):
        csl = pl.ds(j * CHUNK, CHUNK)
        du = pltpu.async_copy(ut_h.at[urow.at[csl]], ubuf, sem)
        dv = pltpu.async_copy(it_h.at[irow.at[csl]], vbuf, sem)
        du.wait()
        dv.wait()

        def group(g, carry):
            rows = g * 16 + iota16
            ucols = ucol[pl.ds(j * CHUNK + g * 16, 16)]
            icols = icol[pl.ds(j * CHUNK + g * 16, 16)]
            acc = bvec
            for d in range(DIM):
                gu = plsc.load_gather(ubuf, [rows, ucols + d])
                gv = plsc.load_gather(vbuf, [rows, icols + d])
                acc = acc + gu * gv * w_s[d]
            y = 1.0 / (1.0 + jnp.exp(-acc))
            outv[pl.ds(j * CHUNK + g * 16, 16)] = y
            return carry

        lax.fori_loop(0, GROUPS, group, 0)

    pltpu.sync_copy(outv, out_h.at[pl.ds(base, ROWS_PER_W)])


@jax.jit
def _gmf(user, item, ut4, it4, w32, b16):
    mesh = plsc.VectorSubcoreMesh(core_axis_name="c", subcore_axis_name="s",
                                  num_cores=N_CORES, num_subcores=N_SUBCORES)
    run = pl.kernel(
        _gmf_body,
        out_type=jax.ShapeDtypeStruct((BATCH,), jnp.float32),
        mesh=mesh,
        scratch_types=[
            pltpu.VMEM((ROWS_PER_W,), jnp.int32),
            pltpu.VMEM((ROWS_PER_W,), jnp.int32),
            pltpu.VMEM((ROWS_PER_W,), jnp.int32),
            pltpu.VMEM((ROWS_PER_W,), jnp.int32),
            pltpu.VMEM((ROWS_PER_W,), jnp.int32),
            pltpu.VMEM((ROWS_PER_W,), jnp.int32),
            pltpu.VMEM((CHUNK, 128), jnp.float32),
            pltpu.VMEM((CHUNK, 128), jnp.float32),
            pltpu.VMEM((DIM,), jnp.float32),
            pltpu.VMEM((16,), jnp.float32),
            pltpu.VMEM((ROWS_PER_W,), jnp.float32),
            pltpu.SemaphoreType.DMA,
        ],
        compiler_params=pltpu.CompilerParams(needs_layout_passes=False),
    )
    return run(user, item, ut4, it4, w32, b16)


def kernel(user, item, users_table, items_table, W, b):
    ut4 = users_table.reshape(-1, 128)
    it4 = items_table.reshape(-1, 128)
    w32 = W.reshape(DIM)
    b16 = jnp.broadcast_to(b.reshape(()), (16,))
    return _gmf(user, item, ut4, it4, w32, b16)


# speedup vs baseline: 1.0194x; 1.0194x over previous
import jax
import jax.numpy as jnp
from jax import lax
from jax.experimental import pallas as pl
from jax.experimental.pallas import tpu as pltpu
from jax.experimental.pallas import tpu_sc as plsc

N_CORES = 2
N_SUBCORES = 16
N_WORKERS = N_CORES * N_SUBCORES
BATCH = 16384
DIM = 32
ROWS_PER_W = BATCH // N_WORKERS
CHUNK = 128
N_CHUNKS = ROWS_PER_W // CHUNK
GROUPS = ROWS_PER_W // 16


def _gmf_body(user_h, item_h, ut_h, it_h, w_h, b_h, out_h,
              uidx, iidx, urows, vrows, wv, bv, outv, sem):
    wid = lax.axis_index("s") * N_CORES + lax.axis_index("c")
    base = wid * ROWS_PER_W

    pltpu.sync_copy(user_h.at[pl.ds(base, ROWS_PER_W)], uidx)
    pltpu.sync_copy(item_h.at[pl.ds(base, ROWS_PER_W)], iidx)
    pltpu.sync_copy(w_h, wv)
    pltpu.sync_copy(b_h, bv)

    descs = []
    for j in range(N_CHUNKS):
        sl = pl.ds(j * CHUNK, CHUNK)
        descs.append(pltpu.async_copy(ut_h.at[uidx.at[sl]], urows.at[sl], sem))
        descs.append(pltpu.async_copy(it_h.at[iidx.at[sl]], vrows.at[sl], sem))
    for d in descs:
        d.wait()

    wlo = wv[pl.ds(0, 16)]
    whi = wv[pl.ds(16, 16)]
    w_s = [wlo[d] for d in range(16)] + [whi[d] for d in range(16)]
    bvec = bv[...]
    iota16 = lax.iota(jnp.int32, 16)

    cols = [jnp.full((16,), d, jnp.int32) for d in range(DIM)]

    def group(g, carry):
        rows = g * 16 + iota16
        acc = bvec
        for d in range(DIM):
            gu = plsc.load_gather(urows, [rows, cols[d]])
            gv = plsc.load_gather(vrows, [rows, cols[d]])
            acc = acc + gu * gv * w_s[d]
        y = 1.0 / (1.0 + jnp.exp(-acc))
        outv[pl.ds(g * 16, 16)] = y
        return carry

    lax.fori_loop(0, GROUPS, group, 0)
    pltpu.sync_copy(outv, out_h.at[pl.ds(base, ROWS_PER_W)])


@jax.jit
def _gmf(user, item, users_table, items_table, w32, b16):
    mesh = plsc.VectorSubcoreMesh(core_axis_name="c", subcore_axis_name="s",
                                  num_cores=N_CORES, num_subcores=N_SUBCORES)
    run = pl.kernel(
        _gmf_body,
        out_type=jax.ShapeDtypeStruct((BATCH,), jnp.float32),
        mesh=mesh,
        scratch_types=[
            pltpu.VMEM((ROWS_PER_W,), jnp.int32),
            pltpu.VMEM((ROWS_PER_W,), jnp.int32),
            pltpu.VMEM((ROWS_PER_W, DIM), jnp.float32),
            pltpu.VMEM((ROWS_PER_W, DIM), jnp.float32),
            pltpu.VMEM((DIM,), jnp.float32),
            pltpu.VMEM((16,), jnp.float32),
            pltpu.VMEM((ROWS_PER_W,), jnp.float32),
            pltpu.SemaphoreType.DMA,
        ],
        compiler_params=pltpu.CompilerParams(needs_layout_passes=False, use_tc_tiling_on_sc=False),
    )
    return run(user, item, users_table, items_table, w32, b16)


def kernel(user, item, users_table, items_table, W, b):
    w32 = W.reshape(DIM)
    b16 = jnp.broadcast_to(b.reshape(()), (16,))
    return _gmf(user, item, users_table, items_table, w32, b16)
